# Initial kernel scaffold; baseline (speedup 1.0000x reference)
import jax
import jax.numpy as jnp
from jax.experimental import pallas as pl


def kernel(x, edge_index, W1, b1, W2, b2):
    raise NotImplementedError("write your pallas kernel here")


# speedup vs baseline: 19.2559x; 19.2559x over previous
import functools

import jax
import jax.numpy as jnp
from jax import lax
from jax.experimental import pallas as pl
from jax.experimental.pallas import tpu as pltpu
from jax.experimental.pallas import tpu_sc as plsc

N = 10000
NP = 10240
E = 320000
F_IN = 128
F_HID = 128
F_OUT = 64

NC = 2
NS = 16
NW = NC * NS
EPT = E // NW
CHUNK = 80
NCHUNK = EPT // CHUNK
RPT = NP // NS
ROWS_N = N // NS

BLK = 1024
GRID = 10

_mesh = plsc.VectorSubcoreMesh(
    core_axis_name="c", subcore_axis_name="s", num_cores=NC, num_subcores=NS
)


def _zero_rows(rows_v, nrow, width):
    zero16 = jnp.zeros((16,), jnp.float32)

    def body(r, carry):
        for c in range(width // 16):
            rows_v[r, pl.ds(c * 16, 16)] = zero16
        return carry

    lax.fori_loop(0, nrow, body, 0)


@functools.partial(
    pl.kernel,
    out_type=jax.ShapeDtypeStruct((NC, NP, 16), jnp.float32),
    mesh=_mesh,
    scratch_types=[
        pltpu.VMEM((NCHUNK, CHUNK), jnp.int32),
        pltpu.VMEM((CHUNK, 16), jnp.float32),
        pltpu.VMEM_SHARED((NP, 16), jnp.float32),
    ],
)
def _deg_kernel(dst4_hbm, deg_out, dst_blk, vals_v, acc_sh):
    cid = lax.axis_index("c")
    sid = lax.axis_index("s")

    _zero_rows(vals_v, CHUNK, 16)
    for k in range(RPT // CHUNK):
        pltpu.sync_copy(vals_v, acc_sh.at[pl.ds(sid * RPT + k * CHUNK, CHUNK)])

    one16 = jnp.ones((16,), jnp.float32)

    def ones_body(r, carry):
        vals_v[r, :] = one16
        return carry

    lax.fori_loop(0, CHUNK, ones_body, 0)
    plsc.subcore_barrier()

    pltpu.sync_copy(dst4_hbm.at[cid, sid], dst_blk)

    def chunk_body(j, carry):
        pltpu.sync_copy(vals_v, acc_sh.at[dst_blk.at[j]], add=True)
        return carry

    lax.fori_loop(0, NCHUNK, chunk_body, 0)

    plsc.subcore_barrier()
    pltpu.sync_copy(
        acc_sh.at[pl.ds(sid * RPT, RPT)], deg_out.at[cid, pl.ds(sid * RPT, RPT)]
    )


def _make_agg_kernel(F):
    @functools.partial(
        pl.kernel,
        out_type=jax.ShapeDtypeStruct((NC, NP, F), jnp.float32),
        mesh=_mesh,
        scratch_types=[
            pltpu.VMEM((NCHUNK, CHUNK), jnp.int32),
            pltpu.VMEM((NCHUNK, CHUNK), jnp.int32),
            pltpu.VMEM((CHUNK, F), jnp.float32),
            pltpu.VMEM_SHARED((NP, F), jnp.float32),
            pltpu.SemaphoreType.DMA,
        ],
    )
    def agg(src4_hbm, dst4_hbm, g_hbm, out_hbm, src_blk, dst_blk, rows_v, acc_sh, sem):
        cid = lax.axis_index("c")
        sid = lax.axis_index("s")

        _zero_rows(rows_v, CHUNK, F)
        for k in range(RPT // CHUNK):
            pltpu.sync_copy(
                rows_v, acc_sh.at[pl.ds(sid * RPT + k * CHUNK, CHUNK)]
            )
        plsc.subcore_barrier()

        pltpu.sync_copy(src4_hbm.at[cid, sid], src_blk)
        pltpu.sync_copy(dst4_hbm.at[cid, sid], dst_blk)

        def chunk_body(j, carry):
            pltpu.async_copy(g_hbm.at[src_blk.at[j]], rows_v, sem).wait()
            pltpu.sync_copy(rows_v, acc_sh.at[dst_blk.at[j]], add=True)
            return carry

        lax.fori_loop(0, NCHUNK, chunk_body, 0)

        plsc.subcore_barrier()
        pltpu.sync_copy(
            acc_sh.at[pl.ds(sid * RPT, RPT)],
            out_hbm.at[cid, pl.ds(sid * RPT, RPT)],
        )

    return agg


_agg128 = _make_agg_kernel(F_HID)


def _mm1_body(x_ref, w_ref, o_ref):
    o_ref[...] = jnp.dot(x_ref[...], w_ref[...], preferred_element_type=jnp.float32)


def _scale_body(h_ref, d0_ref, d1_ref, o_ref):
    dinv = lax.rsqrt(d0_ref[...] + d1_ref[...] + 1.0)
    o_ref[...] = dinv * h_ref[...]


def _mid_body(s_ref, g_ref, d0_ref, d1_ref, b1_ref, o_ref):
    dinv = lax.rsqrt(d0_ref[...] + d1_ref[...] + 1.0)
    a1 = dinv * (s_ref[0] + s_ref[1] + g_ref[...]) + b1_ref[...]
    o_ref[...] = dinv * jnp.maximum(a1, 0.0)


def _final_body(s_ref, p_ref, d0_ref, d1_ref, w2_ref, b2_ref, o_ref):
    dinv = lax.rsqrt(d0_ref[...] + d1_ref[...] + 1.0)
    t = s_ref[0] + s_ref[1] + p_ref[...]
    o_ref[...] = (
        dinv * jnp.dot(t, w2_ref[...], preferred_element_type=jnp.float32)
        + b2_ref[...]
    )


def kernel(x, edge_index, W1, b1, W2, b2):
    src4 = edge_index[0].reshape(NC, NS, NCHUNK, CHUNK)
    dst4 = edge_index[1].reshape(NC, NS, NCHUNK, CHUNK)

    deg_parts = _deg_kernel(dst4)
    d0 = deg_parts[0, :N, 0:1]
    d1 = deg_parts[1, :N, 0:1]

    h1 = pl.pallas_call(
        _mm1_body,
        grid=(GRID,),
        in_specs=[
            pl.BlockSpec((BLK, F_IN), lambda i: (i, 0)),
            pl.BlockSpec((F_IN, F_HID), lambda i: (0, 0)),
        ],
        out_specs=pl.BlockSpec((BLK, F_HID), lambda i: (i, 0)),
        out_shape=jax.ShapeDtypeStruct((N, F_HID), jnp.float32),
    )(x, W1)

    g1 = pl.pallas_call(
        _scale_body,
        grid=(GRID,),
        in_specs=[
            pl.BlockSpec((BLK, F_HID), lambda i: (i, 0)),
            pl.BlockSpec((BLK, 1), lambda i: (i, 0)),
            pl.BlockSpec((BLK, 1), lambda i: (i, 0)),
        ],
        out_specs=pl.BlockSpec((BLK, F_HID), lambda i: (i, 0)),
        out_shape=jax.ShapeDtypeStruct((N, F_HID), jnp.float32),
    )(h1, d0, d1)

    s1 = _agg128(src4, dst4, g1)

    p = pl.pallas_call(
        _mid_body,
        grid=(GRID,),
        in_specs=[
            pl.BlockSpec((NC, BLK, F_HID), lambda i: (0, i, 0)),
            pl.BlockSpec((BLK, F_HID), lambda i: (i, 0)),
            pl.BlockSpec((BLK, 1), lambda i: (i, 0)),
            pl.BlockSpec((BLK, 1), lambda i: (i, 0)),
            pl.BlockSpec((F_HID,), lambda i: (0,)),
        ],
        out_specs=pl.BlockSpec((BLK, F_HID), lambda i: (i, 0)),
        out_shape=jax.ShapeDtypeStruct((N, F_HID), jnp.float32),
    )(s1, g1, d0, d1, b1)

    s2 = _agg128(src4, dst4, p)

    out = pl.pallas_call(
        _final_body,
        grid=(GRID,),
        in_specs=[
            pl.BlockSpec((NC, BLK, F_HID), lambda i: (0, i, 0)),
            pl.BlockSpec((BLK, F_HID), lambda i: (i, 0)),
            pl.BlockSpec((BLK, 1), lambda i: (i, 0)),
            pl.BlockSpec((BLK, 1), lambda i: (i, 0)),
            pl.BlockSpec((F_HID, F_OUT), lambda i: (0, 0)),
            pl.BlockSpec((F_OUT,), lambda i: (0,)),
        ],
        out_specs=pl.BlockSpec((BLK, F_OUT), lambda i: (i, 0)),
        out_shape=jax.ShapeDtypeStruct((N, F_OUT), jnp.float32),
    )(s2, p, d0, d1, W2, b2)

    return out
